# Initial kernel scaffold; baseline (speedup 1.0000x reference)
import jax
import jax.numpy as jnp
from jax.experimental import pallas as pl


def kernel(x, rel_pos_embedding, rel_positions):
    raise NotImplementedError("write your pallas kernel here")


# speedup vs baseline: 4.1285x; 4.1285x over previous
import functools

import jax
import jax.numpy as jnp
from jax import lax
from jax.experimental import pallas as pl
from jax.experimental.pallas import tpu as pltpu
from jax.experimental.pallas import tpu_sc as plsc

NUM_CORES = 2
NUM_SUBCORES = 16
NUM_WORKERS = NUM_CORES * NUM_SUBCORES
CHUNK = 128


def _make_gather(n_rows: int, d: int):
  per_w = n_rows // NUM_WORKERS
  n_chunks = per_w // CHUNK
  mesh = plsc.VectorSubcoreMesh(
      core_axis_name="c", subcore_axis_name="s",
      num_cores=NUM_CORES, num_subcores=NUM_SUBCORES)

  @functools.partial(
      pl.kernel,
      mesh=mesh,
      out_type=jax.ShapeDtypeStruct((n_rows, d), jnp.float32),
      scratch_types=[
          pltpu.VMEM((per_w,), jnp.int32),
          pltpu.VMEM((CHUNK, d), jnp.float32),
          pltpu.SemaphoreType.DMA,
      ],
  )
  def gather_kernel(table_hbm, idx_hbm, out_hbm, idx_v, buf, sem):
    wid = lax.axis_index("s") * NUM_CORES + lax.axis_index("c")
    base = wid * per_w
    pltpu.sync_copy(idx_hbm.at[pl.ds(base, per_w)], idx_v)

    @pl.loop(0, n_chunks)
    def _(c):
      off = c * CHUNK
      pltpu.async_copy(
          table_hbm.at[idx_v.at[pl.ds(off, CHUNK)]], buf, sem).wait()
      pltpu.sync_copy(buf, out_hbm.at[pl.ds(base + off, CHUNK)])

  return gather_kernel


def kernel(x, rel_pos_embedding, rel_positions):
  seq_len = x.shape[1]
  d = rel_pos_embedding.shape[1]
  idx_flat = rel_positions[:seq_len, :seq_len].reshape(-1).astype(jnp.int32)
  gather = _make_gather(idx_flat.shape[0], d)
  rel_pos = gather(rel_pos_embedding, idx_flat)
  return (x, rel_pos.reshape(seq_len, seq_len, d))


# speedup vs baseline: 4.2409x; 1.0272x over previous
import functools

import jax
import jax.numpy as jnp
from jax import lax
from jax.experimental import pallas as pl
from jax.experimental.pallas import tpu as pltpu
from jax.experimental.pallas import tpu_sc as plsc

NUM_CORES = 2
NUM_SUBCORES = 16
NUM_WORKERS = NUM_CORES * NUM_SUBCORES
CHUNK = 128


def _make_gather(n_rows: int, d: int):
  per_w = n_rows // NUM_WORKERS
  n_chunks = per_w // CHUNK
  mesh = plsc.VectorSubcoreMesh(
      core_axis_name="c", subcore_axis_name="s",
      num_cores=NUM_CORES, num_subcores=NUM_SUBCORES)

  assert n_chunks >= 4 and n_chunks % 2 == 0

  @functools.partial(
      pl.kernel,
      mesh=mesh,
      out_type=jax.ShapeDtypeStruct((n_rows, d), jnp.float32),
      scratch_types=[
          pltpu.VMEM((per_w,), jnp.int32),
          pltpu.VMEM((CHUNK, d), jnp.float32),
          pltpu.VMEM((CHUNK, d), jnp.float32),
          pltpu.SemaphoreType.DMA,
          pltpu.SemaphoreType.DMA,
          pltpu.SemaphoreType.DMA,
          pltpu.SemaphoreType.DMA,
      ],
  )
  def gather_kernel(table_hbm, idx_hbm, out_hbm,
                    idx_v, buf0, buf1, gs0, gs1, ws0, ws1):
    wid = lax.axis_index("s") * NUM_CORES + lax.axis_index("c")
    base = wid * per_w
    pltpu.sync_copy(idx_hbm.at[pl.ds(base, per_w)], idx_v)

    def start_gather(c, buf, sem):
      pltpu.async_copy(
          table_hbm.at[idx_v.at[pl.ds(c * CHUNK, CHUNK)]], buf, sem)

    def start_write(c, buf, sem):
      pltpu.async_copy(buf, out_hbm.at[pl.ds(base + c * CHUNK, CHUNK)], sem)

    def wait_gather(buf, sem):
      pltpu.make_async_copy(
          table_hbm.at[idx_v.at[pl.ds(0, CHUNK)]], buf, sem).wait()

    def wait_write(buf, sem):
      pltpu.make_async_copy(buf, out_hbm.at[pl.ds(base, CHUNK)], sem).wait()

    start_gather(0, buf0, gs0)
    start_gather(1, buf1, gs1)
    wait_gather(buf0, gs0)
    start_write(0, buf0, ws0)

    @pl.loop(2, n_chunks, step=2)
    def _(c):
      wait_write(buf0, ws0)
      start_gather(c, buf0, gs0)
      wait_gather(buf1, gs1)
      start_write(c - 1, buf1, ws1)
      wait_write(buf1, ws1)
      start_gather(c + 1, buf1, gs1)
      wait_gather(buf0, gs0)
      start_write(c, buf0, ws0)

    wait_gather(buf1, gs1)
    start_write(n_chunks - 1, buf1, ws1)
    wait_write(buf0, ws0)
    wait_write(buf1, ws1)

  return gather_kernel


def kernel(x, rel_pos_embedding, rel_positions):
  seq_len = x.shape[1]
  d = rel_pos_embedding.shape[1]
  idx_flat = rel_positions[:seq_len, :seq_len].reshape(-1).astype(jnp.int32)
  gather = _make_gather(idx_flat.shape[0], d)
  rel_pos = gather(rel_pos_embedding, idx_flat)
  return (x, rel_pos.reshape(seq_len, seq_len, d))
